# Initial kernel scaffold; baseline (speedup 1.0000x reference)
import jax
import jax.numpy as jnp
from jax.experimental import pallas as pl


def kernel(scalar_features, cartesian_pos, edge_index, W_edge, b_edge, g_edge, be_edge, W_msg, b_msg, g_msg, be_msg, W_upd, b_upd, g_upd, be_upd):
    raise NotImplementedError("write your pallas kernel here")


# speedup vs baseline: 1.7547x; 1.7547x over previous
import functools
import math

import jax
import jax.numpy as jnp
from jax import lax
from jax.experimental import pallas as pl
from jax.experimental.pallas import tpu as pltpu
from jax.experimental.pallas import tpu_sc as plsc

N = 10000
E = 320000
D = 128
MP_STEPS = 3
RBF_DIM = 16
CUTOFF = 10.0
RBF_SUB = 8

NC = 2
NS = 16
NW = NC * NS
PER_W = E // NW
CE = 80
NCHUNK = PER_W // CE
NSTRIPE = N // CE
STR_PER_TILE = (NSTRIPE + NS - 1) // NS

EBLK = 2000
_MESH = plsc.VectorSubcoreMesh(core_axis_name="c", subcore_axis_name="s")


def _gelu(x):
    z = x * 0.7071067811865476
    s = jnp.sign(z)
    az = jnp.abs(z)
    t = 1.0 / (1.0 + 0.3275911 * az)
    poly = ((((1.061405429 * t - 1.453152027) * t) + 1.421413741) * t
            - 0.284496736) * t + 0.254829592
    erf = s * (1.0 - poly * t * jnp.exp(-az * az))
    return 0.5 * x * (1.0 + erf)


def _ln(x, g, b):
    mu = jnp.mean(x, axis=-1, keepdims=True)
    var = jnp.mean((x - mu) ** 2, axis=-1, keepdims=True)
    return (x - mu) / jnp.sqrt(var + 1e-5) * g + b


def _sc_prologue_body(pos16, row, col, zo, posr_o, posc_o, cnt_o,
                      idxr_v, idxc_v, bufr_v, bufc_v, zero_v, ones_v,
                      cnt_sh, sem1, sem2):
    cid = lax.axis_index("c")
    sid = lax.axis_index("s")
    wid = sid * NC + cid
    pltpu.sync_copy(zo.at[0], zero_v)
    pltpu.sync_copy(zo.at[1], ones_v)
    for k in range(STR_PER_TILE):
        c = sid * STR_PER_TILE + k
        @pl.when(c < NSTRIPE)
        def _():
            pltpu.sync_copy(zero_v, cnt_sh.at[pl.ds(c * CE, CE)])
    plsc.subcore_barrier()

    def chunk(k, carry):
        base = wid * PER_W + k * CE
        pltpu.sync_copy(row.at[pl.ds(base, CE)], idxr_v)
        pltpu.sync_copy(col.at[pl.ds(base, CE)], idxc_v)
        cpr = pltpu.async_copy(pos16.at[idxr_v], bufr_v, sem1)
        cpc = pltpu.async_copy(pos16.at[idxc_v], bufc_v, sem2)
        cpr.wait()
        cpc.wait()
        pltpu.sync_copy(bufr_v, posr_o.at[pl.ds(base, CE)])
        pltpu.sync_copy(bufc_v, posc_o.at[pl.ds(base, CE)])
        pltpu.sync_copy(ones_v, cnt_sh.at[idxc_v], add=True)
        return carry

    lax.fori_loop(0, NCHUNK, chunk, 0)
    plsc.subcore_barrier()
    for k in range(STR_PER_TILE):
        c = sid * STR_PER_TILE + k
        @pl.when(c < NSTRIPE)
        def _():
            pltpu.sync_copy(cnt_sh.at[pl.ds(c * CE, CE)], zero_v)
            pltpu.sync_copy(zero_v, cnt_o.at[cid, pl.ds(c * CE, CE)])


def _sc_prologue(pos16, row, col, zo):
    f = functools.partial(
        pl.kernel, _sc_prologue_body, mesh=_MESH,
        out_type=(jax.ShapeDtypeStruct((E, 16), jnp.float32),
                  jax.ShapeDtypeStruct((E, 16), jnp.float32),
                  jax.ShapeDtypeStruct((NC, N, 16), jnp.float32)),
        scratch_types=[
            pltpu.VMEM((CE,), jnp.int32),
            pltpu.VMEM((CE,), jnp.int32),
            pltpu.VMEM((CE, 16), jnp.float32),
            pltpu.VMEM((CE, 16), jnp.float32),
            pltpu.VMEM((CE, 16), jnp.float32),
            pltpu.VMEM((CE, 16), jnp.float32),
            pltpu.VMEM_SHARED((N, 16), jnp.float32),
            pltpu.SemaphoreType.DMA,
            pltpu.SemaphoreType.DMA,
        ],
        compiler_params=pltpu.CompilerParams(use_tc_tiling_on_sc=False))()
    return f(pos16, row, col, zo)


def _sc_gather_body(a_hbm, b_hbm, row, col, ga_o, gb_o,
                    idxr_v, idxc_v, bufa_v, bufb_v, sem1, sem2):
    cid = lax.axis_index("c")
    sid = lax.axis_index("s")
    wid = sid * NC + cid

    def chunk(k, carry):
        base = wid * PER_W + k * CE
        pltpu.sync_copy(row.at[pl.ds(base, CE)], idxr_v)
        pltpu.sync_copy(col.at[pl.ds(base, CE)], idxc_v)
        cpa = pltpu.async_copy(a_hbm.at[idxr_v], bufa_v, sem1)
        cpb = pltpu.async_copy(b_hbm.at[idxc_v], bufb_v, sem2)
        cpa.wait()
        cpb.wait()
        pltpu.sync_copy(bufa_v, ga_o.at[pl.ds(base, CE)])
        pltpu.sync_copy(bufb_v, gb_o.at[pl.ds(base, CE)])
        return carry

    lax.fori_loop(0, NCHUNK, chunk, 0)


def _sc_gather(a, b, row, col):
    f = functools.partial(
        pl.kernel, _sc_gather_body, mesh=_MESH,
        out_type=(jax.ShapeDtypeStruct((E, D), jnp.float32),
                  jax.ShapeDtypeStruct((E, D), jnp.float32)),
        scratch_types=[
            pltpu.VMEM((CE,), jnp.int32),
            pltpu.VMEM((CE,), jnp.int32),
            pltpu.VMEM((CE, D), jnp.float32),
            pltpu.VMEM((CE, D), jnp.float32),
            pltpu.SemaphoreType.DMA,
            pltpu.SemaphoreType.DMA,
        ])()
    return f(a, b, row, col)


def _sc_scatter_body(m_hbm, col, zblk, agg_o,
                     idx_v, buf_v, agg_sh, sem1):
    cid = lax.axis_index("c")
    sid = lax.axis_index("s")
    wid = sid * NC + cid
    pltpu.sync_copy(zblk, buf_v)
    for k in range(STR_PER_TILE):
        c = sid * STR_PER_TILE + k
        @pl.when(c < NSTRIPE)
        def _():
            pltpu.sync_copy(buf_v, agg_sh.at[pl.ds(c * CE, CE)])
    plsc.subcore_barrier()

    def chunk(k, carry):
        base = wid * PER_W + k * CE
        pltpu.sync_copy(col.at[pl.ds(base, CE)], idx_v)
        cpm = pltpu.async_copy(m_hbm.at[pl.ds(base, CE)], buf_v, sem1)
        cpm.wait()
        pltpu.sync_copy(buf_v, agg_sh.at[idx_v], add=True)
        return carry

    lax.fori_loop(0, NCHUNK, chunk, 0)
    plsc.subcore_barrier()
    for k in range(STR_PER_TILE):
        c = sid * STR_PER_TILE + k
        @pl.when(c < NSTRIPE)
        def _():
            pltpu.sync_copy(agg_sh.at[pl.ds(c * CE, CE)], buf_v)
            pltpu.sync_copy(buf_v, agg_o.at[cid, pl.ds(c * CE, CE)])


def _sc_scatter(m, col, zblk):
    f = functools.partial(
        pl.kernel, _sc_scatter_body, mesh=_MESH,
        out_type=jax.ShapeDtypeStruct((NC, N, D), jnp.float32),
        scratch_types=[
            pltpu.VMEM((CE,), jnp.int32),
            pltpu.VMEM((CE, D), jnp.float32),
            pltpu.VMEM_SHARED((N, D), jnp.float32),
            pltpu.SemaphoreType.DMA,
        ])()
    return f(m, col, zblk)


def _edgefeat_body(posr_ref, posc_ref, w_ref, b_ref, g_ref, be_ref, out_ref):
    rel = posr_ref[...] - posc_ref[...]
    x = rel[:, 0:1]
    y = rel[:, 1:2]
    z = rel[:, 2:3]
    d2 = x * x + y * y + z * z
    dist = jnp.sqrt(d2)
    th = dist * (math.pi / CUTOFF)
    s1 = jnp.sin(th)
    c1 = jnp.cos(th)
    two_c1 = 2.0 * c1
    sins = [s1]
    s_prev = jnp.zeros_like(s1)
    s_cur = s1
    for _ in range(RBF_DIM - 1):
        s_nxt = two_c1 * s_cur - s_prev
        sins.append(s_nxt)
        s_prev, s_cur = s_cur, s_nxt
    sinmat = jnp.concatenate(sins, axis=1)
    mask = (dist < CUTOFF).astype(jnp.float32)
    f_cut = 0.5 * (c1 + 1.0) * mask
    rbf = sinmat / (dist + 1e-8) * f_cut
    inv = 1.0 / (dist + 1e-10)
    xd = x * inv
    yd = y * inv
    zd = z * inv
    s0 = 0.2820947917738781 * mask
    sph = [0.4886025119029199 * yd,
           0.4886025119029199 * zd,
           0.4886025119029199 * xd,
           0.5462742152960396 * xd * yd,
           0.5462742152960396 * yd * zd,
           0.6307831305050401 * (3.0 * zd * zd - 1.0) * 0.5,
           0.5462742152960396 * xd * zd,
           0.5462742152960396 * (xd * xd - yd * yd) * 0.5]
    sph = [c * mask for c in sph]
    sphmat = jnp.concatenate(sph, axis=1)
    l0 = rbf * s0
    hi = [rbf[:, bb:bb + 1] * sphmat for bb in range(RBF_SUB)]
    feats = jnp.concatenate([l0] + hi, axis=1)
    zlin = jnp.dot(feats, w_ref[...],
                   preferred_element_type=jnp.float32) + b_ref[...]
    out_ref[...] = _ln(_gelu(zlin), g_ref[...], be_ref[...])


def _tc_edgefeat(posr, posc, w, b, g, be):
    grid = E // EBLK
    return pl.pallas_call(
        _edgefeat_body,
        grid=(grid,),
        in_specs=[
            pl.BlockSpec((EBLK, 16), lambda i: (i, 0)),
            pl.BlockSpec((EBLK, 16), lambda i: (i, 0)),
            pl.BlockSpec((80, D), lambda i: (0, 0)),
            pl.BlockSpec((1, D), lambda i: (0, 0)),
            pl.BlockSpec((1, D), lambda i: (0, 0)),
            pl.BlockSpec((1, D), lambda i: (0, 0)),
        ],
        out_specs=pl.BlockSpec((EBLK, D), lambda i: (i, 0)),
        out_shape=jax.ShapeDtypeStruct((E, D), jnp.float32),
    )(posr, posc, w, b, g, be)


def _nodeproj_body(h_ref, wr_ref, wc_ref, a_ref, b_ref):
    h = h_ref[...]
    a_ref[...] = jnp.dot(h, wr_ref[...], preferred_element_type=jnp.float32)
    b_ref[...] = jnp.dot(h, wc_ref[...], preferred_element_type=jnp.float32)


def _tc_nodeproj(h, wr, wc):
    return pl.pallas_call(
        _nodeproj_body,
        out_shape=(jax.ShapeDtypeStruct((N, D), jnp.float32),
                   jax.ShapeDtypeStruct((N, D), jnp.float32)),
    )(h, wr, wc)


def _edgemath_body(ga_ref, gb_ref, ea_ref, we_ref, b_ref, g_ref, be_ref,
                   m_ref):
    pre = (ga_ref[...] + gb_ref[...]
           + jnp.dot(ea_ref[...], we_ref[...],
                     preferred_element_type=jnp.float32) + b_ref[...])
    m_ref[...] = _ln(_gelu(pre), g_ref[...], be_ref[...])


def _tc_edgemath(ga, gb, ea, we, b, g, be):
    grid = E // EBLK
    return pl.pallas_call(
        _edgemath_body,
        grid=(grid,),
        in_specs=[
            pl.BlockSpec((EBLK, D), lambda i: (i, 0)),
            pl.BlockSpec((EBLK, D), lambda i: (i, 0)),
            pl.BlockSpec((EBLK, D), lambda i: (i, 0)),
            pl.BlockSpec((D, D), lambda i: (0, 0)),
            pl.BlockSpec((1, D), lambda i: (0, 0)),
            pl.BlockSpec((1, D), lambda i: (0, 0)),
            pl.BlockSpec((1, D), lambda i: (0, 0)),
        ],
        out_specs=pl.BlockSpec((EBLK, D), lambda i: (i, 0)),
        out_shape=jax.ShapeDtypeStruct((E, D), jnp.float32),
    )(ga, gb, ea, we, b, g, be)


def _update_body(h_ref, agg_ref, cnt_ref, wu1_ref, wu2_ref, b_ref, g_ref,
                 be_ref, out_ref):
    counts = cnt_ref[0, :, 0:1] + cnt_ref[1, :, 0:1]
    dinv = 1.0 / jnp.maximum(counts, 1.0)
    agg = (agg_ref[0] + agg_ref[1]) * dinv
    h = h_ref[...]
    pre = (jnp.dot(h, wu1_ref[...], preferred_element_type=jnp.float32)
           + jnp.dot(agg, wu2_ref[...], preferred_element_type=jnp.float32)
           + b_ref[...])
    out_ref[...] = h + _ln(pre, g_ref[...], be_ref[...])


def _tc_update(h, agg2, cnt2, wu1, wu2, b, g, be):
    return pl.pallas_call(
        _update_body,
        out_shape=jax.ShapeDtypeStruct((N, D), jnp.float32),
    )(h, agg2, cnt2, wu1, wu2, b, g, be)


def kernel(scalar_features, cartesian_pos, edge_index, W_edge, b_edge,
           g_edge, be_edge, W_msg, b_msg, g_msg, be_msg, W_upd, b_upd,
           g_upd, be_upd):
    row = edge_index[0].astype(jnp.int32)
    col = edge_index[1].astype(jnp.int32)
    pos16 = jnp.zeros((N, 16), jnp.float32).at[:, :3].set(cartesian_pos)
    zo = jnp.stack([jnp.zeros((CE, 16), jnp.float32),
                    jnp.ones((CE, 16), jnp.float32)])
    zblk = jnp.zeros((CE, D), jnp.float32)

    posr, posc, cnt2 = _sc_prologue(pos16, row, col, zo)
    ea = _tc_edgefeat(posr, posc, W_edge,
                      b_edge.reshape(1, D), g_edge.reshape(1, D),
                      be_edge.reshape(1, D))

    h = scalar_features
    for i in range(MP_STEPS):
        wm = W_msg[i]
        a, b = _tc_nodeproj(h, wm[:D], wm[D:2 * D])
        ga, gb = _sc_gather(a, b, row, col)
        m = _tc_edgemath(ga, gb, ea, wm[2 * D:],
                         b_msg[i].reshape(1, D), g_msg[i].reshape(1, D),
                         be_msg[i].reshape(1, D))
        agg2 = _sc_scatter(m, col, zblk)
        wu = W_upd[i]
        h = _tc_update(h, agg2, cnt2, wu[:D], wu[D:],
                       b_upd[i].reshape(1, D), g_upd[i].reshape(1, D),
                       be_upd[i].reshape(1, D))
    return h


# speedup vs baseline: 1.9038x; 1.0849x over previous
import functools
import math

import jax
import jax.numpy as jnp
from jax import lax
from jax.experimental import pallas as pl
from jax.experimental.pallas import tpu as pltpu
from jax.experimental.pallas import tpu_sc as plsc

N = 10000
E = 320000
D = 128
MP_STEPS = 3
RBF_DIM = 16
CUTOFF = 10.0
RBF_SUB = 8

NC = 2
NS = 16
NW = NC * NS
PER_W = E // NW
CE = 80
NCHUNK = PER_W // CE
NSTRIPE = N // CE
STR_PER_TILE = (NSTRIPE + NS - 1) // NS

EBLK = 2000
_MESH = plsc.VectorSubcoreMesh(core_axis_name="c", subcore_axis_name="s")


def _gelu(x):
    z = x * 0.7071067811865476
    s = jnp.sign(z)
    az = jnp.abs(z)
    t = 1.0 / (1.0 + 0.3275911 * az)
    poly = ((((1.061405429 * t - 1.453152027) * t) + 1.421413741) * t
            - 0.284496736) * t + 0.254829592
    erf = s * (1.0 - poly * t * jnp.exp(-az * az))
    return 0.5 * x * (1.0 + erf)


def _ln(x, g, b):
    mu = jnp.mean(x, axis=-1, keepdims=True)
    var = jnp.mean((x - mu) ** 2, axis=-1, keepdims=True)
    return (x - mu) / jnp.sqrt(var + 1e-5) * g + b


def _sc_prologue_body(pos16, row3, col3, zo, posr_o, posc_o, cnt_o,
                      idxr_v, idxc_v, bufr_v, bufc_v, zero_v, ones_v,
                      cnt_sh, sem1, sem2, sst):
    cid = lax.axis_index("c")
    sid = lax.axis_index("s")
    wid = sid * NC + cid
    pltpu.sync_copy(row3.at[wid], idxr_v)
    pltpu.sync_copy(col3.at[wid], idxc_v)
    pltpu.sync_copy(zo.at[0], zero_v)
    pltpu.sync_copy(zo.at[1], ones_v)
    for k in range(STR_PER_TILE):
        c = sid * STR_PER_TILE + k
        @pl.when(c < NSTRIPE)
        def _():
            pltpu.sync_copy(zero_v, cnt_sh.at[pl.ds(c * CE, CE)])
    plsc.subcore_barrier()

    def start_gather(k, s):
        pltpu.async_copy(pos16.at[idxr_v.at[k]], bufr_v.at[s], sem1.at[s])
        pltpu.async_copy(pos16.at[idxc_v.at[k]], bufc_v.at[s], sem2.at[s])

    def wait_gather(k, s):
        pltpu.make_async_copy(pos16.at[idxr_v.at[k]], bufr_v.at[s],
                              sem1.at[s]).wait()
        pltpu.make_async_copy(pos16.at[idxc_v.at[k]], bufc_v.at[s],
                              sem2.at[s]).wait()

    start_gather(0, 0)

    def chunk(k, carry):
        s = lax.rem(k, 2)
        base = wid * PER_W + k * CE

        @pl.when(k + 1 < NCHUNK)
        def _():
            @pl.when(k >= 1)
            def _():
                pltpu.make_async_copy(
                    bufr_v.at[1 - s],
                    posr_o.at[pl.ds(base - CE, CE)], sst.at[1 - s]).wait()
                pltpu.make_async_copy(
                    bufc_v.at[1 - s],
                    posc_o.at[pl.ds(base - CE, CE)], sst.at[1 - s]).wait()
            start_gather(k + 1, 1 - s)

        wait_gather(k, s)
        pltpu.async_copy(bufr_v.at[s], posr_o.at[pl.ds(base, CE)], sst.at[s])
        pltpu.async_copy(bufc_v.at[s], posc_o.at[pl.ds(base, CE)], sst.at[s])
        pltpu.sync_copy(ones_v, cnt_sh.at[idxc_v.at[k]], add=True)
        return carry

    lax.fori_loop(0, NCHUNK, chunk, 0)
    for t in (NCHUNK - 2, NCHUNK - 1):
        base = wid * PER_W + t * CE
        pltpu.make_async_copy(bufr_v.at[t % 2],
                              posr_o.at[pl.ds(base, CE)], sst.at[t % 2]).wait()
        pltpu.make_async_copy(bufc_v.at[t % 2],
                              posc_o.at[pl.ds(base, CE)], sst.at[t % 2]).wait()
    plsc.subcore_barrier()
    for k in range(STR_PER_TILE):
        c = sid * STR_PER_TILE + k
        @pl.when(c < NSTRIPE)
        def _():
            pltpu.sync_copy(cnt_sh.at[pl.ds(c * CE, CE)], zero_v)
            pltpu.sync_copy(zero_v, cnt_o.at[cid, pl.ds(c * CE, CE)])


def _sc_prologue(pos16, row3, col3, zo):
    f = functools.partial(
        pl.kernel, _sc_prologue_body, mesh=_MESH,
        out_type=(jax.ShapeDtypeStruct((E, 16), jnp.float32),
                  jax.ShapeDtypeStruct((E, 16), jnp.float32),
                  jax.ShapeDtypeStruct((NC, N, 16), jnp.float32)),
        scratch_types=[
            pltpu.VMEM((NCHUNK, CE), jnp.int32),
            pltpu.VMEM((NCHUNK, CE), jnp.int32),
            pltpu.VMEM((2, CE, 16), jnp.float32),
            pltpu.VMEM((2, CE, 16), jnp.float32),
            pltpu.VMEM((CE, 16), jnp.float32),
            pltpu.VMEM((CE, 16), jnp.float32),
            pltpu.VMEM_SHARED((N, 16), jnp.float32),
            pltpu.SemaphoreType.DMA((2,)),
            pltpu.SemaphoreType.DMA((2,)),
            pltpu.SemaphoreType.DMA((2,)),
        ],
        compiler_params=pltpu.CompilerParams(use_tc_tiling_on_sc=False))()
    return f(pos16, row3, col3, zo)


def _sc_gather_body(a_hbm, b_hbm, row3, col3, g_o,
                    idxr_v, idxc_v, bufa_v, bufb_v, sga, sgb, sst):
    cid = lax.axis_index("c")
    sid = lax.axis_index("s")
    wid = sid * NC + cid
    pltpu.sync_copy(row3.at[wid], idxr_v)
    pltpu.sync_copy(col3.at[wid], idxc_v)

    def start_gather(k, s):
        pltpu.async_copy(a_hbm.at[idxr_v.at[k]], bufa_v.at[s], sga.at[s])
        pltpu.async_copy(b_hbm.at[idxc_v.at[k]], bufb_v.at[s], sgb.at[s])

    def wait_gather(k, s):
        pltpu.make_async_copy(a_hbm.at[idxr_v.at[k]], bufa_v.at[s],
                              sga.at[s]).wait()
        pltpu.make_async_copy(b_hbm.at[idxc_v.at[k]], bufb_v.at[s],
                              sgb.at[s]).wait()

    def out_slice(k):
        return g_o.at[pl.ds(wid * PER_W + k * CE, CE)]

    start_gather(0, 0)

    def body(k, carry):
        s = lax.rem(k, 2)

        @pl.when(k + 1 < NCHUNK)
        def _():
            @pl.when(k >= 1)
            def _():
                pltpu.make_async_copy(bufa_v.at[1 - s], out_slice(k - 1),
                                      sst.at[1 - s]).wait()
            start_gather(k + 1, 1 - s)

        wait_gather(k, s)

        def add_row(j, c):
            for t in range(D // 16):
                sl = pl.ds(t * 16, 16)
                bufa_v[s, j, sl] = bufa_v[s, j, sl] + bufb_v[s, j, sl]
            return c

        lax.fori_loop(0, CE, add_row, 0)
        pltpu.async_copy(bufa_v.at[s], out_slice(k), sst.at[s])
        return carry

    lax.fori_loop(0, NCHUNK, body, 0)
    for t in (NCHUNK - 2, NCHUNK - 1):
        pltpu.make_async_copy(bufa_v.at[t % 2], out_slice(t),
                              sst.at[t % 2]).wait()


def _sc_gather(a, b, row3, col3):
    f = functools.partial(
        pl.kernel, _sc_gather_body, mesh=_MESH,
        out_type=jax.ShapeDtypeStruct((E, D), jnp.float32),
        scratch_types=[
            pltpu.VMEM((NCHUNK, CE), jnp.int32),
            pltpu.VMEM((NCHUNK, CE), jnp.int32),
            pltpu.VMEM((2, CE, D), jnp.float32),
            pltpu.VMEM((2, CE, D), jnp.float32),
            pltpu.SemaphoreType.DMA((2,)),
            pltpu.SemaphoreType.DMA((2,)),
            pltpu.SemaphoreType.DMA((2,)),
        ])()
    return f(a, b, row3, col3)


def _sc_scatter_body(m_hbm, col3, zblk, agg_o,
                     idx_v, buf_v, agg_sh, sld, sst):
    cid = lax.axis_index("c")
    sid = lax.axis_index("s")
    wid = sid * NC + cid
    pltpu.sync_copy(col3.at[wid], idx_v)
    pltpu.sync_copy(zblk, buf_v.at[0])
    for k in range(STR_PER_TILE):
        c = sid * STR_PER_TILE + k
        @pl.when(c < NSTRIPE)
        def _():
            pltpu.sync_copy(buf_v.at[0], agg_sh.at[pl.ds(c * CE, CE)])
    plsc.subcore_barrier()

    def m_slice(k):
        return m_hbm.at[pl.ds(wid * PER_W + k * CE, CE)]

    pltpu.async_copy(m_slice(0), buf_v.at[0], sld.at[0])

    def body(k, carry):
        s = lax.rem(k, 2)

        @pl.when(k + 1 < NCHUNK)
        def _():
            pltpu.async_copy(m_slice(k + 1), buf_v.at[1 - s], sld.at[1 - s])

        pltpu.make_async_copy(m_slice(k), buf_v.at[s], sld.at[s]).wait()
        pltpu.sync_copy(buf_v.at[s], agg_sh.at[idx_v.at[k]], add=True)
        return carry

    lax.fori_loop(0, NCHUNK, body, 0)
    plsc.subcore_barrier()
    for k in range(STR_PER_TILE):
        c = sid * STR_PER_TILE + k
        s = k % 2
        @pl.when(c < NSTRIPE)
        def _():
            if k >= 2:
                pltpu.make_async_copy(
                    buf_v.at[s],
                    agg_o.at[cid, pl.ds((c - 2) * CE, CE)], sst.at[s]).wait()
            pltpu.sync_copy(agg_sh.at[pl.ds(c * CE, CE)], buf_v.at[s])
            pltpu.async_copy(buf_v.at[s],
                             agg_o.at[cid, pl.ds(c * CE, CE)], sst.at[s])
    for k in range(STR_PER_TILE - 2, STR_PER_TILE):
        c = sid * STR_PER_TILE + k
        @pl.when(c < NSTRIPE)
        def _():
            pltpu.make_async_copy(buf_v.at[k % 2],
                                  agg_o.at[cid, pl.ds(c * CE, CE)],
                                  sst.at[k % 2]).wait()


def _sc_scatter(m, col3, zblk):
    f = functools.partial(
        pl.kernel, _sc_scatter_body, mesh=_MESH,
        out_type=jax.ShapeDtypeStruct((NC, N, D), jnp.float32),
        scratch_types=[
            pltpu.VMEM((NCHUNK, CE), jnp.int32),
            pltpu.VMEM((2, CE, D), jnp.float32),
            pltpu.VMEM_SHARED((N, D), jnp.float32),
            pltpu.SemaphoreType.DMA((2,)),
            pltpu.SemaphoreType.DMA((2,)),
        ])()
    return f(m, col3, zblk)


def _edgefeat_body(posr_ref, posc_ref, w_ref, b_ref, g_ref, be_ref, out_ref):
    rel = posr_ref[...] - posc_ref[...]
    x = rel[:, 0:1]
    y = rel[:, 1:2]
    z = rel[:, 2:3]
    d2 = x * x + y * y + z * z
    dist = jnp.sqrt(d2)
    th = dist * (math.pi / CUTOFF)
    s1 = jnp.sin(th)
    c1 = jnp.cos(th)
    two_c1 = 2.0 * c1
    sins = [s1]
    s_prev = jnp.zeros_like(s1)
    s_cur = s1
    for _ in range(RBF_DIM - 1):
        s_nxt = two_c1 * s_cur - s_prev
        sins.append(s_nxt)
        s_prev, s_cur = s_cur, s_nxt
    sinmat = jnp.concatenate(sins, axis=1)
    mask = (dist < CUTOFF).astype(jnp.float32)
    f_cut = 0.5 * (c1 + 1.0) * mask
    rbf = sinmat / (dist + 1e-8) * f_cut
    inv = 1.0 / (dist + 1e-10)
    xd = x * inv
    yd = y * inv
    zd = z * inv
    s0 = 0.2820947917738781 * mask
    sph = [0.4886025119029199 * yd,
           0.4886025119029199 * zd,
           0.4886025119029199 * xd,
           0.5462742152960396 * xd * yd,
           0.5462742152960396 * yd * zd,
           0.6307831305050401 * (3.0 * zd * zd - 1.0) * 0.5,
           0.5462742152960396 * xd * zd,
           0.5462742152960396 * (xd * xd - yd * yd) * 0.5]
    sph = [c * mask for c in sph]
    sphmat = jnp.concatenate(sph, axis=1)
    l0 = rbf * s0
    hi = [rbf[:, bb:bb + 1] * sphmat for bb in range(RBF_SUB)]
    feats = jnp.concatenate([l0] + hi, axis=1)
    zlin = jnp.dot(feats, w_ref[...],
                   preferred_element_type=jnp.float32) + b_ref[...]
    out_ref[...] = _ln(_gelu(zlin), g_ref[...], be_ref[...])


def _tc_edgefeat(posr, posc, w, b, g, be):
    grid = E // EBLK
    return pl.pallas_call(
        _edgefeat_body,
        grid=(grid,),
        in_specs=[
            pl.BlockSpec((EBLK, 16), lambda i: (i, 0)),
            pl.BlockSpec((EBLK, 16), lambda i: (i, 0)),
            pl.BlockSpec((80, D), lambda i: (0, 0)),
            pl.BlockSpec((1, D), lambda i: (0, 0)),
            pl.BlockSpec((1, D), lambda i: (0, 0)),
            pl.BlockSpec((1, D), lambda i: (0, 0)),
        ],
        out_specs=pl.BlockSpec((EBLK, D), lambda i: (i, 0)),
        out_shape=jax.ShapeDtypeStruct((E, D), jnp.float32),
    )(posr, posc, w, b, g, be)


def _nodeproj_body(h_ref, wr_ref, wc_ref, a_ref, b_ref):
    h = h_ref[...]
    a_ref[...] = jnp.dot(h, wr_ref[...], preferred_element_type=jnp.float32)
    b_ref[...] = jnp.dot(h, wc_ref[...], preferred_element_type=jnp.float32)


def _tc_nodeproj(h, wr, wc):
    return pl.pallas_call(
        _nodeproj_body,
        out_shape=(jax.ShapeDtypeStruct((N, D), jnp.float32),
                   jax.ShapeDtypeStruct((N, D), jnp.float32)),
    )(h, wr, wc)


def _edgemath_body(gab_ref, ea_ref, we_ref, b_ref, g_ref, be_ref,
                   m_ref):
    pre = (gab_ref[...]
           + jnp.dot(ea_ref[...], we_ref[...],
                     preferred_element_type=jnp.float32) + b_ref[...])
    m_ref[...] = _ln(_gelu(pre), g_ref[...], be_ref[...])


def _tc_edgemath(gab, ea, we, b, g, be):
    grid = E // EBLK
    return pl.pallas_call(
        _edgemath_body,
        grid=(grid,),
        in_specs=[
            pl.BlockSpec((EBLK, D), lambda i: (i, 0)),
            pl.BlockSpec((EBLK, D), lambda i: (i, 0)),
            pl.BlockSpec((D, D), lambda i: (0, 0)),
            pl.BlockSpec((1, D), lambda i: (0, 0)),
            pl.BlockSpec((1, D), lambda i: (0, 0)),
            pl.BlockSpec((1, D), lambda i: (0, 0)),
        ],
        out_specs=pl.BlockSpec((EBLK, D), lambda i: (i, 0)),
        out_shape=jax.ShapeDtypeStruct((E, D), jnp.float32),
    )(gab, ea, we, b, g, be)


def _update_body(h_ref, agg_ref, cnt_ref, wu1_ref, wu2_ref, b_ref, g_ref,
                 be_ref, out_ref):
    counts = cnt_ref[0, :, 0:1] + cnt_ref[1, :, 0:1]
    dinv = 1.0 / jnp.maximum(counts, 1.0)
    agg = (agg_ref[0] + agg_ref[1]) * dinv
    h = h_ref[...]
    pre = (jnp.dot(h, wu1_ref[...], preferred_element_type=jnp.float32)
           + jnp.dot(agg, wu2_ref[...], preferred_element_type=jnp.float32)
           + b_ref[...])
    out_ref[...] = h + _ln(pre, g_ref[...], be_ref[...])


def _tc_update(h, agg2, cnt2, wu1, wu2, b, g, be):
    return pl.pallas_call(
        _update_body,
        out_shape=jax.ShapeDtypeStruct((N, D), jnp.float32),
    )(h, agg2, cnt2, wu1, wu2, b, g, be)


def kernel(scalar_features, cartesian_pos, edge_index, W_edge, b_edge,
           g_edge, be_edge, W_msg, b_msg, g_msg, be_msg, W_upd, b_upd,
           g_upd, be_upd):
    row3 = edge_index[0].astype(jnp.int32).reshape(NW, NCHUNK, CE)
    col3 = edge_index[1].astype(jnp.int32).reshape(NW, NCHUNK, CE)
    pos16 = jnp.zeros((N, 16), jnp.float32).at[:, :3].set(cartesian_pos)
    zo = jnp.stack([jnp.zeros((CE, 16), jnp.float32),
                    jnp.ones((CE, 16), jnp.float32)])
    zblk = jnp.zeros((CE, D), jnp.float32)

    posr, posc, cnt2 = _sc_prologue(pos16, row3, col3, zo)
    ea = _tc_edgefeat(posr, posc, W_edge,
                      b_edge.reshape(1, D), g_edge.reshape(1, D),
                      be_edge.reshape(1, D))

    h = scalar_features
    for i in range(MP_STEPS):
        wm = W_msg[i]
        a, b = _tc_nodeproj(h, wm[:D], wm[D:2 * D])
        gab = _sc_gather(a, b, row3, col3)
        m = _tc_edgemath(gab, ea, wm[2 * D:],
                         b_msg[i].reshape(1, D), g_msg[i].reshape(1, D),
                         be_msg[i].reshape(1, D))
        agg2 = _sc_scatter(m, col3, zblk)
        wu = W_upd[i]
        h = _tc_update(h, agg2, cnt2, wu[:D], wu[D:],
                       b_upd[i].reshape(1, D), g_upd[i].reshape(1, D),
                       be_upd[i].reshape(1, D))
    return h


# speedup vs baseline: 3.6463x; 1.9153x over previous
import functools
import math

import jax
import jax.numpy as jnp
from jax import lax
from jax.experimental import pallas as pl
from jax.experimental.pallas import tpu as pltpu
from jax.experimental.pallas import tpu_sc as plsc

N = 10000
E = 320000
D = 128
MP_STEPS = 3
RBF_DIM = 16
CUTOFF = 10.0
RBF_SUB = 8

NC = 2
NS = 16
NW = NC * NS
PER_W = E // NW
CE = 80
NCHUNK = PER_W // CE
NSTRIPE = N // CE
STR_PER_TILE = (NSTRIPE + NS - 1) // NS

EBLK = 2000
EBLK2 = 2560
_MESH = plsc.VectorSubcoreMesh(core_axis_name="c", subcore_axis_name="s")

import numpy as _np
_ASEL = _np.zeros((16, 80), _np.float32)
_BSEL = _np.zeros((16, 80), _np.float32)
for _j in range(16):
    _ASEL[_j, _j] = 1.0
    _BSEL[0, _j] = 1.0
for _b in range(8):
    for _s in range(8):
        _ASEL[_b, 16 + 8 * _b + _s] = 1.0
        _BSEL[1 + _s, 16 + 8 * _b + _s] = 1.0


def _gelu(x):
    z = x * 0.7071067811865476
    s = jnp.sign(z)
    az = jnp.abs(z)
    t = 1.0 / (1.0 + 0.3275911 * az)
    poly = ((((1.061405429 * t - 1.453152027) * t) + 1.421413741) * t
            - 0.284496736) * t + 0.254829592
    erf = s * (1.0 - poly * t * jnp.exp(-az * az))
    return 0.5 * x * (1.0 + erf)


def _ln(x, g, b):
    mu = jnp.mean(x, axis=-1, keepdims=True)
    var = jnp.mean((x - mu) ** 2, axis=-1, keepdims=True)
    return (x - mu) / jnp.sqrt(var + 1e-5) * g + b


def _sc_prologue_body(pos16, row3, col3, zo, relt_o, cnt_o,
                      idxr_v, idxc_v, bufr_v, bufc_v, rel_v, buft_v,
                      zero_v, ones_v, cnt_sh, sem1, sem2, sst):
    cid = lax.axis_index("c")
    sid = lax.axis_index("s")
    wid = sid * NC + cid
    pltpu.sync_copy(row3.at[wid], idxr_v)
    pltpu.sync_copy(col3.at[wid], idxc_v)
    pltpu.sync_copy(zo.at[0], zero_v)
    pltpu.sync_copy(zo.at[1], ones_v)
    zvec = jnp.zeros((16,), jnp.float32)
    for st in range(2):
        for r in range(8):
            for g in range(CE // 16):
                buft_v[st, r, pl.ds(g * 16, 16)] = zvec
    for k in range(STR_PER_TILE):
        c = sid * STR_PER_TILE + k
        @pl.when(c < NSTRIPE)
        def _():
            pltpu.sync_copy(zero_v, cnt_sh.at[pl.ds(c * CE, CE)])
    plsc.subcore_barrier()

    def start_gather(k, s):
        pltpu.async_copy(pos16.at[idxr_v.at[k]], bufr_v.at[s], sem1.at[s])
        pltpu.async_copy(pos16.at[idxc_v.at[k]], bufc_v.at[s], sem2.at[s])

    def wait_gather(k, s):
        pltpu.make_async_copy(pos16.at[idxr_v.at[k]], bufr_v.at[s],
                              sem1.at[s]).wait()
        pltpu.make_async_copy(pos16.at[idxc_v.at[k]], bufc_v.at[s],
                              sem2.at[s]).wait()

    def relt_slice(k):
        return relt_o.at[:, pl.ds(wid * PER_W + k * CE, CE)]

    iota16 = lax.iota(jnp.int32, 16)
    start_gather(0, 0)

    def chunk(k, carry):
        s = lax.rem(k, 2)

        @pl.when(k + 1 < NCHUNK)
        def _():
            @pl.when(k >= 1)
            def _():
                pltpu.make_async_copy(buft_v.at[1 - s], relt_slice(k - 1),
                                      sst.at[1 - s]).wait()
            start_gather(k + 1, 1 - s)

        wait_gather(k, s)

        def sub_row(j, c):
            rel_v[j, :] = bufr_v[s, j, :] - bufc_v[s, j, :]
            return c

        lax.fori_loop(0, CE, sub_row, 0)
        for g in range(CE // 16):
            ridx = iota16 + (g * 16)
            for comp in range(3):
                cidx = jnp.full((16,), comp, jnp.int32)
                v = plsc.load_gather(rel_v, [ridx, cidx])
                buft_v[s, comp, pl.ds(g * 16, 16)] = v
        pltpu.async_copy(buft_v.at[s], relt_slice(k), sst.at[s])
        pltpu.sync_copy(ones_v, cnt_sh.at[idxc_v.at[k]], add=True)
        return carry

    lax.fori_loop(0, NCHUNK, chunk, 0)
    for t in (NCHUNK - 2, NCHUNK - 1):
        pltpu.make_async_copy(buft_v.at[t % 2], relt_slice(t),
                              sst.at[t % 2]).wait()
    plsc.subcore_barrier()
    for k in range(STR_PER_TILE):
        c = sid * STR_PER_TILE + k
        @pl.when(c < NSTRIPE)
        def _():
            pltpu.sync_copy(cnt_sh.at[pl.ds(c * CE, CE)], zero_v)
            pltpu.sync_copy(zero_v, cnt_o.at[cid, pl.ds(c * CE, CE)])


def _sc_prologue(pos16, row3, col3, zo):
    f = functools.partial(
        pl.kernel, _sc_prologue_body, mesh=_MESH,
        out_type=(jax.ShapeDtypeStruct((8, E), jnp.float32),
                  jax.ShapeDtypeStruct((NC, N, 16), jnp.float32)),
        scratch_types=[
            pltpu.VMEM((NCHUNK, CE), jnp.int32),
            pltpu.VMEM((NCHUNK, CE), jnp.int32),
            pltpu.VMEM((2, CE, 16), jnp.float32),
            pltpu.VMEM((2, CE, 16), jnp.float32),
            pltpu.VMEM((CE, 16), jnp.float32),
            pltpu.VMEM((2, 8, CE), jnp.float32),
            pltpu.VMEM((CE, 16), jnp.float32),
            pltpu.VMEM((CE, 16), jnp.float32),
            pltpu.VMEM_SHARED((N, 16), jnp.float32),
            pltpu.SemaphoreType.DMA((2,)),
            pltpu.SemaphoreType.DMA((2,)),
            pltpu.SemaphoreType.DMA((2,)),
        ],
        compiler_params=pltpu.CompilerParams(use_tc_tiling_on_sc=False,
                                             needs_layout_passes=False))()
    return f(pos16, row3, col3, zo)


def _sc_gather_body(a_hbm, b_hbm, row3, col3, g_o,
                    idxr_v, idxc_v, bufa_v, bufb_v, sga, sgb, sst):
    cid = lax.axis_index("c")
    sid = lax.axis_index("s")
    wid = sid * NC + cid
    pltpu.sync_copy(row3.at[wid], idxr_v)
    pltpu.sync_copy(col3.at[wid], idxc_v)

    def start_a(k, s):
        pltpu.async_copy(a_hbm.at[idxr_v.at[k]], bufa_v.at[s], sga.at[s])

    def wait_a(k, s):
        pltpu.make_async_copy(a_hbm.at[idxr_v.at[k]], bufa_v.at[s],
                              sga.at[s]).wait()

    def start_b(k, s):
        pltpu.async_copy(b_hbm.at[idxc_v.at[k]], bufb_v.at[s], sgb.at[s])

    def wait_b(k, s):
        pltpu.make_async_copy(b_hbm.at[idxc_v.at[k]], bufb_v.at[s],
                              sgb.at[s]).wait()

    def out_slice(k):
        return g_o.at[pl.ds(wid * PER_W + k * CE, CE)]

    start_a(0, 0)
    start_b(0, 0)

    def body(k, carry):
        s = lax.rem(k, 2)

        @pl.when(k + 1 < NCHUNK)
        def _():
            @pl.when(k >= 1)
            def _():
                pltpu.make_async_copy(bufa_v.at[1 - s], out_slice(k - 1),
                                      sst.at[1 - s]).wait()
            start_a(k + 1, 1 - s)
            start_b(k + 1, 1 - s)

        wait_a(k, s)
        wait_b(k, s)

        def add_row(j, c):
            for t in range(D // 16):
                sl = pl.ds(t * 16, 16)
                bufa_v[s, j, sl] = bufa_v[s, j, sl] + bufb_v[s, j, sl]
            return c

        lax.fori_loop(0, CE, add_row, 0)
        pltpu.async_copy(bufa_v.at[s], out_slice(k), sst.at[s])
        return carry

    lax.fori_loop(0, NCHUNK, body, 0)
    for t in (NCHUNK - 2, NCHUNK - 1):
        pltpu.make_async_copy(bufa_v.at[t % 2], out_slice(t),
                              sst.at[t % 2]).wait()


def _sc_gather(a, b, row3, col3):
    f = functools.partial(
        pl.kernel, _sc_gather_body, mesh=_MESH,
        out_type=jax.ShapeDtypeStruct((E, D), jnp.float32),
        scratch_types=[
            pltpu.VMEM((NCHUNK, CE), jnp.int32),
            pltpu.VMEM((NCHUNK, CE), jnp.int32),
            pltpu.VMEM((2, CE, D), jnp.float32),
            pltpu.VMEM((2, CE, D), jnp.float32),
            pltpu.SemaphoreType.DMA((2,)),
            pltpu.SemaphoreType.DMA((2,)),
            pltpu.SemaphoreType.DMA((2,)),
        ])()
    return f(a, b, row3, col3)


def _sc_scatter_body(m_hbm, col3, zblk, agg_o,
                     idx_v, buf_v, agg_sh, sld, sst):
    cid = lax.axis_index("c")
    sid = lax.axis_index("s")
    wid = sid * NC + cid
    pltpu.sync_copy(col3.at[wid], idx_v)
    pltpu.sync_copy(zblk, buf_v.at[0])
    for k in range(STR_PER_TILE):
        c = sid * STR_PER_TILE + k
        @pl.when(c < NSTRIPE)
        def _():
            pltpu.sync_copy(buf_v.at[0], agg_sh.at[pl.ds(c * CE, CE)])
    plsc.subcore_barrier()

    def m_slice(k):
        return m_hbm.at[pl.ds(wid * PER_W + k * CE, CE)]

    pltpu.async_copy(m_slice(0), buf_v.at[0], sld.at[0])

    def body(k, carry):
        s = lax.rem(k, 2)

        @pl.when(k + 1 < NCHUNK)
        def _():
            pltpu.async_copy(m_slice(k + 1), buf_v.at[1 - s], sld.at[1 - s])

        pltpu.make_async_copy(m_slice(k), buf_v.at[s], sld.at[s]).wait()
        pltpu.sync_copy(buf_v.at[s], agg_sh.at[idx_v.at[k]], add=True)
        return carry

    lax.fori_loop(0, NCHUNK, body, 0)
    plsc.subcore_barrier()
    for k in range(STR_PER_TILE):
        c = sid * STR_PER_TILE + k
        s = k % 2
        @pl.when(c < NSTRIPE)
        def _():
            if k >= 2:
                pltpu.make_async_copy(
                    buf_v.at[s],
                    agg_o.at[cid, pl.ds((c - 2) * CE, CE)], sst.at[s]).wait()
            pltpu.sync_copy(agg_sh.at[pl.ds(c * CE, CE)], buf_v.at[s])
            pltpu.async_copy(buf_v.at[s],
                             agg_o.at[cid, pl.ds(c * CE, CE)], sst.at[s])
    for k in range(STR_PER_TILE - 2, STR_PER_TILE):
        c = sid * STR_PER_TILE + k
        @pl.when(c < NSTRIPE)
        def _():
            pltpu.make_async_copy(buf_v.at[k % 2],
                                  agg_o.at[cid, pl.ds(c * CE, CE)],
                                  sst.at[k % 2]).wait()


def _sc_scatter(m, col3, zblk):
    f = functools.partial(
        pl.kernel, _sc_scatter_body, mesh=_MESH,
        out_type=jax.ShapeDtypeStruct((NC, N, D), jnp.float32),
        scratch_types=[
            pltpu.VMEM((NCHUNK, CE), jnp.int32),
            pltpu.VMEM((2, CE, D), jnp.float32),
            pltpu.VMEM_SHARED((N, D), jnp.float32),
            pltpu.SemaphoreType.DMA((2,)),
            pltpu.SemaphoreType.DMA((2,)),
        ])()
    return f(m, col3, zblk)


def _edgefeat_body(relt_ref, asel_ref, bsel_ref, w_ref, b_ref, g_ref, be_ref,
                   out_ref):
    x = relt_ref[0:1, :]
    y = relt_ref[1:2, :]
    z = relt_ref[2:3, :]
    d2 = x * x + y * y + z * z
    dist = jnp.sqrt(d2)
    mask = (dist < CUTOFF).astype(jnp.float32)
    th1 = dist * (math.pi / CUTOFF)
    c1 = jnp.cos(th1)
    f_cut = 0.5 * (c1 + 1.0) * mask
    scale = f_cut / (dist + 1e-8)
    kvec = (lax.broadcasted_iota(jnp.int32, (16, 1), 0) + 1
            ).astype(jnp.float32)
    th16 = kvec * th1
    rbf = jnp.sin(th16) * scale
    inv = 1.0 / (dist + 1e-10)
    xd = x * inv
    yd = y * inv
    zd = z * inv
    rows = [0.2820947917738781 * jnp.ones_like(xd),
            0.4886025119029199 * yd,
            0.4886025119029199 * zd,
            0.4886025119029199 * xd,
            0.5462742152960396 * xd * yd,
            0.5462742152960396 * yd * zd,
            0.6307831305050401 * (3.0 * zd * zd - 1.0) * 0.5,
            0.5462742152960396 * xd * zd,
            0.5462742152960396 * (xd * xd - yd * yd) * 0.5,
            jnp.zeros((7, x.shape[1]), jnp.float32)]
    sph = jnp.concatenate(rows, axis=0) * mask
    dn = (((0,), (0,)), ((), ()))
    t1 = lax.dot_general(asel_ref[...], rbf, dn,
                         preferred_element_type=jnp.float32)
    t2 = lax.dot_general(bsel_ref[...], sph, dn,
                         preferred_element_type=jnp.float32)
    feats_t = t1 * t2
    pre = lax.dot_general(feats_t, w_ref[...], dn,
                          preferred_element_type=jnp.float32) + b_ref[...]
    out_ref[...] = _ln(_gelu(pre), g_ref[...], be_ref[...])


def _tc_edgefeat(relt, asel, bsel, w, b, g, be):
    grid = E // EBLK2
    return pl.pallas_call(
        _edgefeat_body,
        grid=(grid,),
        in_specs=[
            pl.BlockSpec((8, EBLK2), lambda i: (0, i)),
            pl.BlockSpec((16, 80), lambda i: (0, 0)),
            pl.BlockSpec((16, 80), lambda i: (0, 0)),
            pl.BlockSpec((80, D), lambda i: (0, 0)),
            pl.BlockSpec((1, D), lambda i: (0, 0)),
            pl.BlockSpec((1, D), lambda i: (0, 0)),
            pl.BlockSpec((1, D), lambda i: (0, 0)),
        ],
        out_specs=pl.BlockSpec((EBLK2, D), lambda i: (i, 0)),
        out_shape=jax.ShapeDtypeStruct((E, D), jnp.float32),
    )(relt, asel, bsel, w, b, g, be)


def _nodeproj_body(h_ref, wr_ref, wc_ref, a_ref, b_ref):
    h = h_ref[...]
    a_ref[...] = jnp.dot(h, wr_ref[...], preferred_element_type=jnp.float32)
    b_ref[...] = jnp.dot(h, wc_ref[...], preferred_element_type=jnp.float32)


def _tc_nodeproj(h, wr, wc):
    return pl.pallas_call(
        _nodeproj_body,
        out_shape=(jax.ShapeDtypeStruct((N, D), jnp.float32),
                   jax.ShapeDtypeStruct((N, D), jnp.float32)),
    )(h, wr, wc)


def _edgemath_body(gab_ref, ea_ref, we_ref, b_ref, g_ref, be_ref,
                   m_ref):
    pre = (gab_ref[...]
           + jnp.dot(ea_ref[...], we_ref[...],
                     preferred_element_type=jnp.float32) + b_ref[...])
    m_ref[...] = _ln(_gelu(pre), g_ref[...], be_ref[...])


def _tc_edgemath(gab, ea, we, b, g, be):
    grid = E // EBLK
    return pl.pallas_call(
        _edgemath_body,
        grid=(grid,),
        in_specs=[
            pl.BlockSpec((EBLK, D), lambda i: (i, 0)),
            pl.BlockSpec((EBLK, D), lambda i: (i, 0)),
            pl.BlockSpec((D, D), lambda i: (0, 0)),
            pl.BlockSpec((1, D), lambda i: (0, 0)),
            pl.BlockSpec((1, D), lambda i: (0, 0)),
            pl.BlockSpec((1, D), lambda i: (0, 0)),
        ],
        out_specs=pl.BlockSpec((EBLK, D), lambda i: (i, 0)),
        out_shape=jax.ShapeDtypeStruct((E, D), jnp.float32),
    )(gab, ea, we, b, g, be)


def _update_body(h_ref, agg_ref, cnt_ref, wu1_ref, wu2_ref, b_ref, g_ref,
                 be_ref, out_ref):
    counts = cnt_ref[0, :, 0:1] + cnt_ref[1, :, 0:1]
    dinv = 1.0 / jnp.maximum(counts, 1.0)
    agg = (agg_ref[0] + agg_ref[1]) * dinv
    h = h_ref[...]
    pre = (jnp.dot(h, wu1_ref[...], preferred_element_type=jnp.float32)
           + jnp.dot(agg, wu2_ref[...], preferred_element_type=jnp.float32)
           + b_ref[...])
    out_ref[...] = h + _ln(pre, g_ref[...], be_ref[...])


def _tc_update(h, agg2, cnt2, wu1, wu2, b, g, be):
    return pl.pallas_call(
        _update_body,
        out_shape=jax.ShapeDtypeStruct((N, D), jnp.float32),
    )(h, agg2, cnt2, wu1, wu2, b, g, be)


def kernel(scalar_features, cartesian_pos, edge_index, W_edge, b_edge,
           g_edge, be_edge, W_msg, b_msg, g_msg, be_msg, W_upd, b_upd,
           g_upd, be_upd):
    row3 = edge_index[0].astype(jnp.int32).reshape(NW, NCHUNK, CE)
    col3 = edge_index[1].astype(jnp.int32).reshape(NW, NCHUNK, CE)
    pos16 = jnp.zeros((N, 16), jnp.float32).at[:, :3].set(cartesian_pos)
    zo = jnp.stack([jnp.zeros((CE, 16), jnp.float32),
                    jnp.ones((CE, 16), jnp.float32)])
    zblk = jnp.zeros((CE, D), jnp.float32)

    relt, cnt2 = _sc_prologue(pos16, row3, col3, zo)
    ea = _tc_edgefeat(relt, jnp.asarray(_ASEL), jnp.asarray(_BSEL), W_edge,
                      b_edge.reshape(1, D), g_edge.reshape(1, D),
                      be_edge.reshape(1, D))

    h = scalar_features
    for i in range(MP_STEPS):
        wm = W_msg[i]
        a, b = _tc_nodeproj(h, wm[:D], wm[D:2 * D])
        gab = _sc_gather(a, b, row3, col3)
        m = _tc_edgemath(gab, ea, wm[2 * D:],
                         b_msg[i].reshape(1, D), g_msg[i].reshape(1, D),
                         be_msg[i].reshape(1, D))
        agg2 = _sc_scatter(m, col3, zblk)
        wu = W_upd[i]
        h = _tc_update(h, agg2, cnt2, wu[:D], wu[D:],
                       b_upd[i].reshape(1, D), g_upd[i].reshape(1, D),
                       be_upd[i].reshape(1, D))
    return h


# speedup vs baseline: 4.5930x; 1.2596x over previous
import functools
import math

import jax
import jax.numpy as jnp
from jax import lax
from jax.experimental import pallas as pl
from jax.experimental.pallas import tpu as pltpu
from jax.experimental.pallas import tpu_sc as plsc

N = 10000
E = 320000
D = 128
MP_STEPS = 3
RBF_DIM = 16
CUTOFF = 10.0
RBF_SUB = 8

NC = 2
NS = 16
NW = NC * NS
PER_W = E // NW
CE = 80
NCHUNK = PER_W // CE
NSTRIPE = N // CE
STR_PER_TILE = (NSTRIPE + NS - 1) // NS

EBLK = 2000
EBLK2 = 2560
_MESH = plsc.VectorSubcoreMesh(core_axis_name="c", subcore_axis_name="s")

import numpy as _np
_ASEL = _np.zeros((16, 80), _np.float32)
_BSEL = _np.zeros((16, 80), _np.float32)
for _j in range(16):
    _ASEL[_j, _j] = 1.0
    _BSEL[0, _j] = 1.0
for _b in range(8):
    for _s in range(8):
        _ASEL[_b, 16 + 8 * _b + _s] = 1.0
        _BSEL[1 + _s, 16 + 8 * _b + _s] = 1.0


def _gelu(x):
    z = x * 0.7071067811865476
    s = jnp.sign(z)
    az = jnp.abs(z)
    t = 1.0 / (1.0 + 0.3275911 * az)
    poly = ((((1.061405429 * t - 1.453152027) * t) + 1.421413741) * t
            - 0.284496736) * t + 0.254829592
    erf = s * (1.0 - poly * t * jnp.exp(-az * az))
    return 0.5 * x * (1.0 + erf)


def _ln(x, g, b):
    mu = jnp.mean(x, axis=-1, keepdims=True)
    var = jnp.mean((x - mu) ** 2, axis=-1, keepdims=True)
    return (x - mu) / jnp.sqrt(var + 1e-5) * g + b


def _sc_prologue_body(pos16, row3, col3, zo, relt_o, cnt_o,
                      idxr_v, idxc_v, bufr_v, bufc_v, rel_v, buft_v,
                      zero_v, ones_v, cnt_sh, sem1, sem2, sst):
    cid = lax.axis_index("c")
    sid = lax.axis_index("s")
    wid = sid * NC + cid
    pltpu.sync_copy(row3.at[wid], idxr_v)
    pltpu.sync_copy(col3.at[wid], idxc_v)
    pltpu.sync_copy(zo.at[0], zero_v)
    pltpu.sync_copy(zo.at[1], ones_v)
    zvec = jnp.zeros((16,), jnp.float32)
    for st in range(2):
        for r in range(8):
            for g in range(CE // 16):
                buft_v[st, r, pl.ds(g * 16, 16)] = zvec
    for k in range(STR_PER_TILE):
        c = sid * STR_PER_TILE + k
        @pl.when(c < NSTRIPE)
        def _():
            pltpu.sync_copy(zero_v, cnt_sh.at[pl.ds(c * CE, CE)])
    plsc.subcore_barrier()

    def start_gather(k, s):
        pltpu.async_copy(pos16.at[idxr_v.at[k]], bufr_v.at[s], sem1.at[s])
        pltpu.async_copy(pos16.at[idxc_v.at[k]], bufc_v.at[s], sem2.at[s])

    def wait_gather(k, s):
        pltpu.make_async_copy(pos16.at[idxr_v.at[k]], bufr_v.at[s],
                              sem1.at[s]).wait()
        pltpu.make_async_copy(pos16.at[idxc_v.at[k]], bufc_v.at[s],
                              sem2.at[s]).wait()

    def relt_slice(k):
        return relt_o.at[:, pl.ds(wid * PER_W + k * CE, CE)]

    iota16 = lax.iota(jnp.int32, 16)
    start_gather(0, 0)

    def chunk(k, carry):
        s = lax.rem(k, 2)

        @pl.when(k + 1 < NCHUNK)
        def _():
            @pl.when(k >= 1)
            def _():
                pltpu.make_async_copy(buft_v.at[1 - s], relt_slice(k - 1),
                                      sst.at[1 - s]).wait()
            start_gather(k + 1, 1 - s)

        wait_gather(k, s)

        def sub_row(j, c):
            rel_v[j, :] = bufr_v[s, j, :] - bufc_v[s, j, :]
            return c

        lax.fori_loop(0, CE, sub_row, 0)
        for g in range(CE // 16):
            ridx = iota16 + (g * 16)
            for comp in range(3):
                cidx = jnp.full((16,), comp, jnp.int32)
                v = plsc.load_gather(rel_v, [ridx, cidx])
                buft_v[s, comp, pl.ds(g * 16, 16)] = v
        pltpu.async_copy(buft_v.at[s], relt_slice(k), sst.at[s])
        pltpu.sync_copy(ones_v, cnt_sh.at[idxc_v.at[k]], add=True)
        return carry

    lax.fori_loop(0, NCHUNK, chunk, 0)
    for t in (NCHUNK - 2, NCHUNK - 1):
        pltpu.make_async_copy(buft_v.at[t % 2], relt_slice(t),
                              sst.at[t % 2]).wait()
    plsc.subcore_barrier()
    for k in range(STR_PER_TILE):
        c = sid * STR_PER_TILE + k
        @pl.when(c < NSTRIPE)
        def _():
            pltpu.sync_copy(cnt_sh.at[pl.ds(c * CE, CE)], zero_v)
            pltpu.sync_copy(zero_v, cnt_o.at[cid, pl.ds(c * CE, CE)])


def _sc_prologue(pos16, row3, col3, zo):
    f = functools.partial(
        pl.kernel, _sc_prologue_body, mesh=_MESH,
        out_type=(jax.ShapeDtypeStruct((8, E), jnp.float32),
                  jax.ShapeDtypeStruct((NC, N, 16), jnp.float32)),
        scratch_types=[
            pltpu.VMEM((NCHUNK, CE), jnp.int32),
            pltpu.VMEM((NCHUNK, CE), jnp.int32),
            pltpu.VMEM((2, CE, 16), jnp.float32),
            pltpu.VMEM((2, CE, 16), jnp.float32),
            pltpu.VMEM((CE, 16), jnp.float32),
            pltpu.VMEM((2, 8, CE), jnp.float32),
            pltpu.VMEM((CE, 16), jnp.float32),
            pltpu.VMEM((CE, 16), jnp.float32),
            pltpu.VMEM_SHARED((N, 16), jnp.float32),
            pltpu.SemaphoreType.DMA((2,)),
            pltpu.SemaphoreType.DMA((2,)),
            pltpu.SemaphoreType.DMA((2,)),
        ],
        compiler_params=pltpu.CompilerParams(use_tc_tiling_on_sc=False,
                                             needs_layout_passes=False))()
    return f(pos16, row3, col3, zo)


def _sc_gather_body(a_hbm, b_hbm, row3, col3, ga_o, gb_o,
                    idxr_v, idxc_v, bufa_v, bufb_v, sga, sgb, sst, sstb):
    cid = lax.axis_index("c")
    sid = lax.axis_index("s")
    wid = sid * NC + cid
    pltpu.sync_copy(row3.at[wid], idxr_v)
    pltpu.sync_copy(col3.at[wid], idxc_v)

    def start_a(k, s):
        pltpu.async_copy(a_hbm.at[idxr_v.at[k]], bufa_v.at[s], sga.at[s])

    def wait_a(k, s):
        pltpu.make_async_copy(a_hbm.at[idxr_v.at[k]], bufa_v.at[s],
                              sga.at[s]).wait()

    def start_b(k, s):
        pltpu.async_copy(b_hbm.at[idxc_v.at[k]], bufb_v.at[s], sgb.at[s])

    def wait_b(k, s):
        pltpu.make_async_copy(b_hbm.at[idxc_v.at[k]], bufb_v.at[s],
                              sgb.at[s]).wait()

    def outa_slice(k):
        return ga_o.at[pl.ds(wid * PER_W + k * CE, CE)]

    def outb_slice(k):
        return gb_o.at[pl.ds(wid * PER_W + k * CE, CE)]

    start_a(0, 0)
    start_b(0, 0)

    def body(k, carry):
        s = lax.rem(k, 2)

        @pl.when(k + 1 < NCHUNK)
        def _():
            @pl.when(k >= 1)
            def _():
                pltpu.make_async_copy(bufa_v.at[1 - s], outa_slice(k - 1),
                                      sst.at[1 - s]).wait()
                pltpu.make_async_copy(bufb_v.at[1 - s], outb_slice(k - 1),
                                      sstb.at[1 - s]).wait()
            start_a(k + 1, 1 - s)
            start_b(k + 1, 1 - s)

        wait_a(k, s)
        wait_b(k, s)
        pltpu.async_copy(bufa_v.at[s], outa_slice(k), sst.at[s])
        pltpu.async_copy(bufb_v.at[s], outb_slice(k), sstb.at[s])
        return carry

    lax.fori_loop(0, NCHUNK, body, 0)
    for t in (NCHUNK - 2, NCHUNK - 1):
        pltpu.make_async_copy(bufa_v.at[t % 2], outa_slice(t),
                              sst.at[t % 2]).wait()
        pltpu.make_async_copy(bufb_v.at[t % 2], outb_slice(t),
                              sstb.at[t % 2]).wait()


def _sc_gather(a, b, row3, col3):
    f = functools.partial(
        pl.kernel, _sc_gather_body, mesh=_MESH,
        out_type=(jax.ShapeDtypeStruct((E, D), jnp.float32),
                  jax.ShapeDtypeStruct((E, D), jnp.float32)),
        scratch_types=[
            pltpu.VMEM((NCHUNK, CE), jnp.int32),
            pltpu.VMEM((NCHUNK, CE), jnp.int32),
            pltpu.VMEM((2, CE, D), jnp.float32),
            pltpu.VMEM((2, CE, D), jnp.float32),
            pltpu.SemaphoreType.DMA((2,)),
            pltpu.SemaphoreType.DMA((2,)),
            pltpu.SemaphoreType.DMA((2,)),
            pltpu.SemaphoreType.DMA((2,)),
        ])()
    return f(a, b, row3, col3)


def _sc_scatter_body(m_hbm, col3, zblk, agg_o,
                     idx_v, buf_v, agg_sh, sld, sst):
    cid = lax.axis_index("c")
    sid = lax.axis_index("s")
    wid = sid * NC + cid
    pltpu.sync_copy(col3.at[wid], idx_v)
    pltpu.sync_copy(zblk, buf_v.at[0])
    for k in range(STR_PER_TILE):
        c = sid * STR_PER_TILE + k
        @pl.when(c < NSTRIPE)
        def _():
            pltpu.sync_copy(buf_v.at[0], agg_sh.at[pl.ds(c * CE, CE)])
    plsc.subcore_barrier()

    def m_slice(k):
        return m_hbm.at[pl.ds(wid * PER_W + k * CE, CE)]

    pltpu.async_copy(m_slice(0), buf_v.at[0], sld.at[0])

    def body(k, carry):
        s = lax.rem(k, 2)

        @pl.when(k + 1 < NCHUNK)
        def _():
            pltpu.async_copy(m_slice(k + 1), buf_v.at[1 - s], sld.at[1 - s])

        pltpu.make_async_copy(m_slice(k), buf_v.at[s], sld.at[s]).wait()
        pltpu.sync_copy(buf_v.at[s], agg_sh.at[idx_v.at[k]], add=True)
        return carry

    lax.fori_loop(0, NCHUNK, body, 0)
    plsc.subcore_barrier()
    for k in range(STR_PER_TILE):
        c = sid * STR_PER_TILE + k
        s = k % 2
        @pl.when(c < NSTRIPE)
        def _():
            if k >= 2:
                pltpu.make_async_copy(
                    buf_v.at[s],
                    agg_o.at[cid, pl.ds((c - 2) * CE, CE)], sst.at[s]).wait()
            pltpu.sync_copy(agg_sh.at[pl.ds(c * CE, CE)], buf_v.at[s])
            pltpu.async_copy(buf_v.at[s],
                             agg_o.at[cid, pl.ds(c * CE, CE)], sst.at[s])
    for k in range(STR_PER_TILE - 2, STR_PER_TILE):
        c = sid * STR_PER_TILE + k
        @pl.when(c < NSTRIPE)
        def _():
            pltpu.make_async_copy(buf_v.at[k % 2],
                                  agg_o.at[cid, pl.ds(c * CE, CE)],
                                  sst.at[k % 2]).wait()


def _sc_scatter(m, col3, zblk):
    f = functools.partial(
        pl.kernel, _sc_scatter_body, mesh=_MESH,
        out_type=jax.ShapeDtypeStruct((NC, N, D), jnp.float32),
        scratch_types=[
            pltpu.VMEM((NCHUNK, CE), jnp.int32),
            pltpu.VMEM((2, CE, D), jnp.float32),
            pltpu.VMEM_SHARED((N, D), jnp.float32),
            pltpu.SemaphoreType.DMA((2,)),
            pltpu.SemaphoreType.DMA((2,)),
        ])()
    return f(m, col3, zblk)


def _edgefeat_body(relt_ref, asel_ref, bsel_ref, w_ref, b_ref, g_ref, be_ref,
                   out_ref):
    x = relt_ref[0:1, :]
    y = relt_ref[1:2, :]
    z = relt_ref[2:3, :]
    d2 = x * x + y * y + z * z
    dist = jnp.sqrt(d2)
    mask = (dist < CUTOFF).astype(jnp.float32)
    th1 = dist * (math.pi / CUTOFF)
    c1 = jnp.cos(th1)
    f_cut = 0.5 * (c1 + 1.0) * mask
    scale = f_cut / (dist + 1e-8)
    kvec = (lax.broadcasted_iota(jnp.int32, (16, 1), 0) + 1
            ).astype(jnp.float32)
    th16 = kvec * th1
    rbf = jnp.sin(th16) * scale
    inv = 1.0 / (dist + 1e-10)
    xd = x * inv
    yd = y * inv
    zd = z * inv
    rows = [0.2820947917738781 * jnp.ones_like(xd),
            0.4886025119029199 * yd,
            0.4886025119029199 * zd,
            0.4886025119029199 * xd,
            0.5462742152960396 * xd * yd,
            0.5462742152960396 * yd * zd,
            0.6307831305050401 * (3.0 * zd * zd - 1.0) * 0.5,
            0.5462742152960396 * xd * zd,
            0.5462742152960396 * (xd * xd - yd * yd) * 0.5,
            jnp.zeros((7, x.shape[1]), jnp.float32)]
    sph = jnp.concatenate(rows, axis=0) * mask
    dn = (((0,), (0,)), ((), ()))
    t1 = lax.dot_general(asel_ref[...], rbf, dn,
                         preferred_element_type=jnp.float32)
    t2 = lax.dot_general(bsel_ref[...], sph, dn,
                         preferred_element_type=jnp.float32)
    feats_t = t1 * t2
    pre = lax.dot_general(feats_t, w_ref[...], dn,
                          preferred_element_type=jnp.float32) + b_ref[...]
    out_ref[...] = _ln(_gelu(pre), g_ref[...], be_ref[...])


def _tc_edgefeat(relt, asel, bsel, w, b, g, be):
    grid = E // EBLK2
    return pl.pallas_call(
        _edgefeat_body,
        grid=(grid,),
        in_specs=[
            pl.BlockSpec((8, EBLK2), lambda i: (0, i)),
            pl.BlockSpec((16, 80), lambda i: (0, 0)),
            pl.BlockSpec((16, 80), lambda i: (0, 0)),
            pl.BlockSpec((80, D), lambda i: (0, 0)),
            pl.BlockSpec((1, D), lambda i: (0, 0)),
            pl.BlockSpec((1, D), lambda i: (0, 0)),
            pl.BlockSpec((1, D), lambda i: (0, 0)),
        ],
        out_specs=pl.BlockSpec((EBLK2, D), lambda i: (i, 0)),
        out_shape=jax.ShapeDtypeStruct((E, D), jnp.float32),
    )(relt, asel, bsel, w, b, g, be)


def _nodeproj_body(h_ref, wr_ref, wc_ref, a_ref, b_ref):
    h = h_ref[...]
    a_ref[...] = jnp.dot(h, wr_ref[...], preferred_element_type=jnp.float32)
    b_ref[...] = jnp.dot(h, wc_ref[...], preferred_element_type=jnp.float32)


def _tc_nodeproj(h, wr, wc):
    return pl.pallas_call(
        _nodeproj_body,
        out_shape=(jax.ShapeDtypeStruct((N, D), jnp.float32),
                   jax.ShapeDtypeStruct((N, D), jnp.float32)),
    )(h, wr, wc)


def _edgemath_body(ga_ref, gb_ref, ea_ref, we_ref, b_ref, g_ref, be_ref,
                   m_ref):
    pre = (ga_ref[...] + gb_ref[...]
           + jnp.dot(ea_ref[...], we_ref[...],
                     preferred_element_type=jnp.float32) + b_ref[...])
    m_ref[...] = _ln(_gelu(pre), g_ref[...], be_ref[...])


def _tc_edgemath(ga, gb, ea, we, b, g, be):
    grid = E // EBLK
    return pl.pallas_call(
        _edgemath_body,
        grid=(grid,),
        in_specs=[
            pl.BlockSpec((EBLK, D), lambda i: (i, 0)),
            pl.BlockSpec((EBLK, D), lambda i: (i, 0)),
            pl.BlockSpec((EBLK, D), lambda i: (i, 0)),
            pl.BlockSpec((D, D), lambda i: (0, 0)),
            pl.BlockSpec((1, D), lambda i: (0, 0)),
            pl.BlockSpec((1, D), lambda i: (0, 0)),
            pl.BlockSpec((1, D), lambda i: (0, 0)),
        ],
        out_specs=pl.BlockSpec((EBLK, D), lambda i: (i, 0)),
        out_shape=jax.ShapeDtypeStruct((E, D), jnp.float32),
    )(ga, gb, ea, we, b, g, be)


def _update_body(h_ref, agg_ref, cnt_ref, wu1_ref, wu2_ref, b_ref, g_ref,
                 be_ref, out_ref):
    counts = cnt_ref[0, :, 0:1] + cnt_ref[1, :, 0:1]
    dinv = 1.0 / jnp.maximum(counts, 1.0)
    agg = (agg_ref[0] + agg_ref[1]) * dinv
    h = h_ref[...]
    pre = (jnp.dot(h, wu1_ref[...], preferred_element_type=jnp.float32)
           + jnp.dot(agg, wu2_ref[...], preferred_element_type=jnp.float32)
           + b_ref[...])
    out_ref[...] = h + _ln(pre, g_ref[...], be_ref[...])


def _tc_update(h, agg2, cnt2, wu1, wu2, b, g, be):
    return pl.pallas_call(
        _update_body,
        out_shape=jax.ShapeDtypeStruct((N, D), jnp.float32),
    )(h, agg2, cnt2, wu1, wu2, b, g, be)


def kernel(scalar_features, cartesian_pos, edge_index, W_edge, b_edge,
           g_edge, be_edge, W_msg, b_msg, g_msg, be_msg, W_upd, b_upd,
           g_upd, be_upd):
    row3 = edge_index[0].astype(jnp.int32).reshape(NW, NCHUNK, CE)
    col3 = edge_index[1].astype(jnp.int32).reshape(NW, NCHUNK, CE)
    pos16 = jnp.zeros((N, 16), jnp.float32).at[:, :3].set(cartesian_pos)
    zo = jnp.stack([jnp.zeros((CE, 16), jnp.float32),
                    jnp.ones((CE, 16), jnp.float32)])
    zblk = jnp.zeros((CE, D), jnp.float32)

    relt, cnt2 = _sc_prologue(pos16, row3, col3, zo)
    ea = _tc_edgefeat(relt, jnp.asarray(_ASEL), jnp.asarray(_BSEL), W_edge,
                      b_edge.reshape(1, D), g_edge.reshape(1, D),
                      be_edge.reshape(1, D))

    h = scalar_features
    for i in range(MP_STEPS):
        wm = W_msg[i]
        a, b = _tc_nodeproj(h, wm[:D], wm[D:2 * D])
        ga, gb = _sc_gather(a, b, row3, col3)
        m = _tc_edgemath(ga, gb, ea, wm[2 * D:],
                         b_msg[i].reshape(1, D), g_msg[i].reshape(1, D),
                         be_msg[i].reshape(1, D))
        agg2 = _sc_scatter(m, col3, zblk)
        wu = W_upd[i]
        h = _tc_update(h, agg2, cnt2, wu[:D], wu[D:],
                       b_upd[i].reshape(1, D), g_upd[i].reshape(1, D),
                       be_upd[i].reshape(1, D))
    return h


# speedup vs baseline: 4.6324x; 1.0086x over previous
import functools
import math

import jax
import jax.numpy as jnp
from jax import lax
from jax.experimental import pallas as pl
from jax.experimental.pallas import tpu as pltpu
from jax.experimental.pallas import tpu_sc as plsc

N = 10000
E = 320000
D = 128
MP_STEPS = 3
RBF_DIM = 16
CUTOFF = 10.0
RBF_SUB = 8

NC = 2
NS = 16
NW = NC * NS
PER_W = E // NW
CE = 80
NCHUNK = PER_W // CE
NSTRIPE = N // CE
STR_PER_TILE = (NSTRIPE + NS - 1) // NS

EBLK = 2000
EBLK2 = 2560
_MESH = plsc.VectorSubcoreMesh(core_axis_name="c", subcore_axis_name="s")

import numpy as _np
_ASEL = _np.zeros((16, 80), _np.float32)
_BSEL = _np.zeros((16, 80), _np.float32)
for _j in range(16):
    _ASEL[_j, _j] = 1.0
    _BSEL[0, _j] = 1.0
for _b in range(8):
    for _s in range(8):
        _ASEL[_b, 16 + 8 * _b + _s] = 1.0
        _BSEL[1 + _s, 16 + 8 * _b + _s] = 1.0


def _gelu(x):
    z = x * 0.7071067811865476
    s = jnp.sign(z)
    az = jnp.abs(z)
    t = 1.0 / (1.0 + 0.3275911 * az)
    poly = ((((1.061405429 * t - 1.453152027) * t) + 1.421413741) * t
            - 0.284496736) * t + 0.254829592
    erf = s * (1.0 - poly * t * jnp.exp(-az * az))
    return 0.5 * x * (1.0 + erf)


def _ln(x, g, b):
    mu = jnp.mean(x, axis=-1, keepdims=True)
    var = jnp.mean((x - mu) ** 2, axis=-1, keepdims=True)
    return (x - mu) / jnp.sqrt(var + 1e-5) * g + b


def _sc_prologue_body(pos16, row3, col3, zo, relt_o, cnt_o,
                      idxr_v, idxc_v, bufr_v, bufc_v, buft_v,
                      zero_v, ones_v, cnt_sh, sem1, sem2, sst):
    cid = lax.axis_index("c")
    sid = lax.axis_index("s")
    wid = sid * NC + cid
    pltpu.sync_copy(row3.at[wid], idxr_v)
    pltpu.sync_copy(col3.at[wid], idxc_v)
    pltpu.sync_copy(zo.at[0], zero_v)
    pltpu.sync_copy(zo.at[1], ones_v)
    zvec = jnp.zeros((16,), jnp.float32)
    for st in range(2):
        for r in range(8):
            for g in range(CE // 16):
                buft_v[st, r, pl.ds(g * 16, 16)] = zvec
    for k in range(STR_PER_TILE):
        c = sid * STR_PER_TILE + k
        @pl.when(c < NSTRIPE)
        def _():
            pltpu.sync_copy(zero_v, cnt_sh.at[pl.ds(c * CE, CE)])
    plsc.subcore_barrier()

    def start_gather(k, s):
        pltpu.async_copy(pos16.at[idxr_v.at[k]], bufr_v.at[s], sem1.at[s])
        pltpu.async_copy(pos16.at[idxc_v.at[k]], bufc_v.at[s], sem2.at[s])

    def wait_gather(k, s):
        pltpu.make_async_copy(pos16.at[idxr_v.at[k]], bufr_v.at[s],
                              sem1.at[s]).wait()
        pltpu.make_async_copy(pos16.at[idxc_v.at[k]], bufc_v.at[s],
                              sem2.at[s]).wait()

    def relt_slice(k):
        return relt_o.at[:, pl.ds(wid * PER_W + k * CE, CE)]

    iota16 = lax.iota(jnp.int32, 16)
    start_gather(0, 0)

    def chunk(k, carry):
        s = lax.rem(k, 2)

        @pl.when(k + 1 < NCHUNK)
        def _():
            @pl.when(k >= 1)
            def _():
                pltpu.make_async_copy(buft_v.at[1 - s], relt_slice(k - 1),
                                      sst.at[1 - s]).wait()
            start_gather(k + 1, 1 - s)

        wait_gather(k, s)
        sfull = jnp.full((16,), s, jnp.int32)
        for g in range(CE // 16):
            ridx = iota16 + (g * 16)
            for comp in range(3):
                cidx = jnp.full((16,), comp, jnp.int32)
                vr = plsc.load_gather(bufr_v, [sfull, ridx, cidx])
                vc = plsc.load_gather(bufc_v, [sfull, ridx, cidx])
                buft_v[s, comp, pl.ds(g * 16, 16)] = vr - vc
        pltpu.async_copy(buft_v.at[s], relt_slice(k), sst.at[s])
        pltpu.sync_copy(ones_v, cnt_sh.at[idxc_v.at[k]], add=True)
        return carry

    lax.fori_loop(0, NCHUNK, chunk, 0)
    for t in (NCHUNK - 2, NCHUNK - 1):
        pltpu.make_async_copy(buft_v.at[t % 2], relt_slice(t),
                              sst.at[t % 2]).wait()
    plsc.subcore_barrier()
    for k in range(STR_PER_TILE):
        c = sid * STR_PER_TILE + k
        @pl.when(c < NSTRIPE)
        def _():
            pltpu.sync_copy(cnt_sh.at[pl.ds(c * CE, CE)], zero_v)
            pltpu.sync_copy(zero_v, cnt_o.at[cid, pl.ds(c * CE, CE)])


def _sc_prologue(pos16, row3, col3, zo):
    f = functools.partial(
        pl.kernel, _sc_prologue_body, mesh=_MESH,
        out_type=(jax.ShapeDtypeStruct((8, E), jnp.float32),
                  jax.ShapeDtypeStruct((NC, N, 16), jnp.float32)),
        scratch_types=[
            pltpu.VMEM((NCHUNK, CE), jnp.int32),
            pltpu.VMEM((NCHUNK, CE), jnp.int32),
            pltpu.VMEM((2, CE, 16), jnp.float32),
            pltpu.VMEM((2, CE, 16), jnp.float32),
            pltpu.VMEM((2, 8, CE), jnp.float32),
            pltpu.VMEM((CE, 16), jnp.float32),
            pltpu.VMEM((CE, 16), jnp.float32),
            pltpu.VMEM_SHARED((N, 16), jnp.float32),
            pltpu.SemaphoreType.DMA((2,)),
            pltpu.SemaphoreType.DMA((2,)),
            pltpu.SemaphoreType.DMA((2,)),
        ],
        compiler_params=pltpu.CompilerParams(use_tc_tiling_on_sc=False,
                                             needs_layout_passes=False))()
    return f(pos16, row3, col3, zo)


def _sc_gather_body(a_hbm, b_hbm, row3, col3, ga_o, gb_o,
                    idxr_v, idxc_v, bufa_v, bufb_v, sga, sgb, sst, sstb):
    cid = lax.axis_index("c")
    sid = lax.axis_index("s")
    wid = sid * NC + cid
    pltpu.sync_copy(row3.at[wid], idxr_v)
    pltpu.sync_copy(col3.at[wid], idxc_v)

    def start_a(k, s):
        pltpu.async_copy(a_hbm.at[idxr_v.at[k]], bufa_v.at[s], sga.at[s])

    def wait_a(k, s):
        pltpu.make_async_copy(a_hbm.at[idxr_v.at[k]], bufa_v.at[s],
                              sga.at[s]).wait()

    def start_b(k, s):
        pltpu.async_copy(b_hbm.at[idxc_v.at[k]], bufb_v.at[s], sgb.at[s])

    def wait_b(k, s):
        pltpu.make_async_copy(b_hbm.at[idxc_v.at[k]], bufb_v.at[s],
                              sgb.at[s]).wait()

    def outa_slice(k):
        return ga_o.at[pl.ds(wid * PER_W + k * CE, CE)]

    def outb_slice(k):
        return gb_o.at[pl.ds(wid * PER_W + k * CE, CE)]

    start_a(0, 0)
    start_b(0, 0)
    start_a(1, 1)
    start_b(1, 1)

    def body(k, carry):
        s = lax.rem(k, 3)

        @pl.when(k + 2 < NCHUNK)
        def _():
            s2 = lax.rem(k + 2, 3)

            @pl.when(k >= 1)
            def _():
                pltpu.make_async_copy(bufa_v.at[s2], outa_slice(k - 1),
                                      sst.at[s2]).wait()
                pltpu.make_async_copy(bufb_v.at[s2], outb_slice(k - 1),
                                      sstb.at[s2]).wait()
            start_a(k + 2, s2)
            start_b(k + 2, s2)

        wait_a(k, s)
        wait_b(k, s)
        pltpu.async_copy(bufa_v.at[s], outa_slice(k), sst.at[s])
        pltpu.async_copy(bufb_v.at[s], outb_slice(k), sstb.at[s])
        return carry

    lax.fori_loop(0, NCHUNK, body, 0)
    for t in (NCHUNK - 3, NCHUNK - 2, NCHUNK - 1):
        pltpu.make_async_copy(bufa_v.at[t % 3], outa_slice(t),
                              sst.at[t % 3]).wait()
        pltpu.make_async_copy(bufb_v.at[t % 3], outb_slice(t),
                              sstb.at[t % 3]).wait()


def _sc_gather(a, b, row3, col3):
    f = functools.partial(
        pl.kernel, _sc_gather_body, mesh=_MESH,
        out_type=(jax.ShapeDtypeStruct((E, D), jnp.float32),
                  jax.ShapeDtypeStruct((E, D), jnp.float32)),
        scratch_types=[
            pltpu.VMEM((NCHUNK, CE), jnp.int32),
            pltpu.VMEM((NCHUNK, CE), jnp.int32),
            pltpu.VMEM((3, CE, D), jnp.float32),
            pltpu.VMEM((3, CE, D), jnp.float32),
            pltpu.SemaphoreType.DMA((3,)),
            pltpu.SemaphoreType.DMA((3,)),
            pltpu.SemaphoreType.DMA((3,)),
            pltpu.SemaphoreType.DMA((3,)),
        ])()
    return f(a, b, row3, col3)


def _sc_scatter_body(m_hbm, col3, zblk, agg_o,
                     idx_v, buf_v, agg_sh, sld, sst):
    cid = lax.axis_index("c")
    sid = lax.axis_index("s")
    wid = sid * NC + cid
    pltpu.sync_copy(col3.at[wid], idx_v)
    pltpu.sync_copy(zblk, buf_v.at[0])
    for k in range(STR_PER_TILE):
        c = sid * STR_PER_TILE + k
        @pl.when(c < NSTRIPE)
        def _():
            pltpu.sync_copy(buf_v.at[0], agg_sh.at[pl.ds(c * CE, CE)])
    plsc.subcore_barrier()

    def m_slice(k):
        return m_hbm.at[pl.ds(wid * PER_W + k * CE, CE)]

    pltpu.async_copy(m_slice(0), buf_v.at[0], sld.at[0])

    def body(k, carry):
        s = lax.rem(k, 2)

        @pl.when(k + 1 < NCHUNK)
        def _():
            pltpu.async_copy(m_slice(k + 1), buf_v.at[1 - s], sld.at[1 - s])

        pltpu.make_async_copy(m_slice(k), buf_v.at[s], sld.at[s]).wait()
        pltpu.sync_copy(buf_v.at[s], agg_sh.at[idx_v.at[k]], add=True)
        return carry

    lax.fori_loop(0, NCHUNK, body, 0)
    plsc.subcore_barrier()
    for k in range(STR_PER_TILE):
        c = sid * STR_PER_TILE + k
        s = k % 2
        @pl.when(c < NSTRIPE)
        def _():
            if k >= 2:
                pltpu.make_async_copy(
                    buf_v.at[s],
                    agg_o.at[cid, pl.ds((c - 2) * CE, CE)], sst.at[s]).wait()
            pltpu.sync_copy(agg_sh.at[pl.ds(c * CE, CE)], buf_v.at[s])
            pltpu.async_copy(buf_v.at[s],
                             agg_o.at[cid, pl.ds(c * CE, CE)], sst.at[s])
    for k in range(STR_PER_TILE - 2, STR_PER_TILE):
        c = sid * STR_PER_TILE + k
        @pl.when(c < NSTRIPE)
        def _():
            pltpu.make_async_copy(buf_v.at[k % 2],
                                  agg_o.at[cid, pl.ds(c * CE, CE)],
                                  sst.at[k % 2]).wait()


def _sc_scatter(m, col3, zblk):
    f = functools.partial(
        pl.kernel, _sc_scatter_body, mesh=_MESH,
        out_type=jax.ShapeDtypeStruct((NC, N, D), jnp.float32),
        scratch_types=[
            pltpu.VMEM((NCHUNK, CE), jnp.int32),
            pltpu.VMEM((2, CE, D), jnp.float32),
            pltpu.VMEM_SHARED((N, D), jnp.float32),
            pltpu.SemaphoreType.DMA((2,)),
            pltpu.SemaphoreType.DMA((2,)),
        ])()
    return f(m, col3, zblk)


def _edgefeat_body(relt_ref, asel_ref, bsel_ref, w_ref, b_ref, g_ref, be_ref,
                   out_ref):
    x = relt_ref[0:1, :]
    y = relt_ref[1:2, :]
    z = relt_ref[2:3, :]
    d2 = x * x + y * y + z * z
    dist = jnp.sqrt(d2)
    mask = (dist < CUTOFF).astype(jnp.float32)
    th1 = dist * (math.pi / CUTOFF)
    c1 = jnp.cos(th1)
    f_cut = 0.5 * (c1 + 1.0) * mask
    scale = f_cut / (dist + 1e-8)
    kvec = (lax.broadcasted_iota(jnp.int32, (16, 1), 0) + 1
            ).astype(jnp.float32)
    th16 = kvec * th1
    rbf = jnp.sin(th16) * scale
    inv = 1.0 / (dist + 1e-10)
    xd = x * inv
    yd = y * inv
    zd = z * inv
    rows = [0.2820947917738781 * jnp.ones_like(xd),
            0.4886025119029199 * yd,
            0.4886025119029199 * zd,
            0.4886025119029199 * xd,
            0.5462742152960396 * xd * yd,
            0.5462742152960396 * yd * zd,
            0.6307831305050401 * (3.0 * zd * zd - 1.0) * 0.5,
            0.5462742152960396 * xd * zd,
            0.5462742152960396 * (xd * xd - yd * yd) * 0.5,
            jnp.zeros((7, x.shape[1]), jnp.float32)]
    sph = jnp.concatenate(rows, axis=0) * mask
    dn = (((0,), (0,)), ((), ()))
    t1 = lax.dot_general(asel_ref[...], rbf, dn,
                         preferred_element_type=jnp.float32)
    t2 = lax.dot_general(bsel_ref[...], sph, dn,
                         preferred_element_type=jnp.float32)
    feats_t = t1 * t2
    pre = lax.dot_general(feats_t, w_ref[...], dn,
                          preferred_element_type=jnp.float32) + b_ref[...]
    out_ref[...] = _ln(_gelu(pre), g_ref[...], be_ref[...])


def _tc_edgefeat(relt, asel, bsel, w, b, g, be):
    grid = E // EBLK2
    return pl.pallas_call(
        _edgefeat_body,
        grid=(grid,),
        in_specs=[
            pl.BlockSpec((8, EBLK2), lambda i: (0, i)),
            pl.BlockSpec((16, 80), lambda i: (0, 0)),
            pl.BlockSpec((16, 80), lambda i: (0, 0)),
            pl.BlockSpec((80, D), lambda i: (0, 0)),
            pl.BlockSpec((1, D), lambda i: (0, 0)),
            pl.BlockSpec((1, D), lambda i: (0, 0)),
            pl.BlockSpec((1, D), lambda i: (0, 0)),
        ],
        out_specs=pl.BlockSpec((EBLK2, D), lambda i: (i, 0)),
        out_shape=jax.ShapeDtypeStruct((E, D), jnp.float32),
    )(relt, asel, bsel, w, b, g, be)


def _nodeproj_body(h_ref, wr_ref, wc_ref, a_ref, b_ref):
    h = h_ref[...]
    a_ref[...] = jnp.dot(h, wr_ref[...], preferred_element_type=jnp.float32)
    b_ref[...] = jnp.dot(h, wc_ref[...], preferred_element_type=jnp.float32)


def _tc_nodeproj(h, wr, wc):
    return pl.pallas_call(
        _nodeproj_body,
        out_shape=(jax.ShapeDtypeStruct((N, D), jnp.float32),
                   jax.ShapeDtypeStruct((N, D), jnp.float32)),
    )(h, wr, wc)


def _edgemath_body(ga_ref, gb_ref, ea_ref, we_ref, b_ref, g_ref, be_ref,
                   m_ref):
    pre = (ga_ref[...] + gb_ref[...]
           + jnp.dot(ea_ref[...], we_ref[...],
                     preferred_element_type=jnp.float32) + b_ref[...])
    m_ref[...] = _ln(_gelu(pre), g_ref[...], be_ref[...])


def _tc_edgemath(ga, gb, ea, we, b, g, be):
    grid = E // EBLK
    return pl.pallas_call(
        _edgemath_body,
        grid=(grid,),
        in_specs=[
            pl.BlockSpec((EBLK, D), lambda i: (i, 0)),
            pl.BlockSpec((EBLK, D), lambda i: (i, 0)),
            pl.BlockSpec((EBLK, D), lambda i: (i, 0)),
            pl.BlockSpec((D, D), lambda i: (0, 0)),
            pl.BlockSpec((1, D), lambda i: (0, 0)),
            pl.BlockSpec((1, D), lambda i: (0, 0)),
            pl.BlockSpec((1, D), lambda i: (0, 0)),
        ],
        out_specs=pl.BlockSpec((EBLK, D), lambda i: (i, 0)),
        out_shape=jax.ShapeDtypeStruct((E, D), jnp.float32),
    )(ga, gb, ea, we, b, g, be)


def _update_body(h_ref, agg_ref, cnt_ref, wu1_ref, wu2_ref, b_ref, g_ref,
                 be_ref, out_ref):
    counts = cnt_ref[0, :, 0:1] + cnt_ref[1, :, 0:1]
    dinv = 1.0 / jnp.maximum(counts, 1.0)
    agg = (agg_ref[0] + agg_ref[1]) * dinv
    h = h_ref[...]
    pre = (jnp.dot(h, wu1_ref[...], preferred_element_type=jnp.float32)
           + jnp.dot(agg, wu2_ref[...], preferred_element_type=jnp.float32)
           + b_ref[...])
    out_ref[...] = h + _ln(pre, g_ref[...], be_ref[...])


def _tc_update(h, agg2, cnt2, wu1, wu2, b, g, be):
    return pl.pallas_call(
        _update_body,
        out_shape=jax.ShapeDtypeStruct((N, D), jnp.float32),
    )(h, agg2, cnt2, wu1, wu2, b, g, be)


def kernel(scalar_features, cartesian_pos, edge_index, W_edge, b_edge,
           g_edge, be_edge, W_msg, b_msg, g_msg, be_msg, W_upd, b_upd,
           g_upd, be_upd):
    row3 = edge_index[0].astype(jnp.int32).reshape(NW, NCHUNK, CE)
    col3 = edge_index[1].astype(jnp.int32).reshape(NW, NCHUNK, CE)
    pos16 = jnp.zeros((N, 16), jnp.float32).at[:, :3].set(cartesian_pos)
    zo = jnp.stack([jnp.zeros((CE, 16), jnp.float32),
                    jnp.ones((CE, 16), jnp.float32)])
    zblk = jnp.zeros((CE, D), jnp.float32)

    relt, cnt2 = _sc_prologue(pos16, row3, col3, zo)
    ea = _tc_edgefeat(relt, jnp.asarray(_ASEL), jnp.asarray(_BSEL), W_edge,
                      b_edge.reshape(1, D), g_edge.reshape(1, D),
                      be_edge.reshape(1, D))

    h = scalar_features
    for i in range(MP_STEPS):
        wm = W_msg[i]
        a, b = _tc_nodeproj(h, wm[:D], wm[D:2 * D])
        ga, gb = _sc_gather(a, b, row3, col3)
        m = _tc_edgemath(ga, gb, ea, wm[2 * D:],
                         b_msg[i].reshape(1, D), g_msg[i].reshape(1, D),
                         be_msg[i].reshape(1, D))
        agg2 = _sc_scatter(m, col3, zblk)
        wu = W_upd[i]
        h = _tc_update(h, agg2, cnt2, wu[:D], wu[D:],
                       b_upd[i].reshape(1, D), g_upd[i].reshape(1, D),
                       be_upd[i].reshape(1, D))
    return h
